# Initial kernel scaffold; baseline (speedup 1.0000x reference)
import jax
import jax.numpy as jnp
from jax.experimental import pallas as pl


def kernel(input_data, edge_index_rel0, edge_index_rel1, W_init, b_init, W1_0, b1_0, W1_1, b1_1, W2_0, b2_0, W2_1, b2_1, W3_0, b3_0, W3_1, b3_1):
    raise NotImplementedError("write your pallas kernel here")


# speedup vs baseline: 5.6384x; 5.6384x over previous
import functools

import jax
import jax.numpy as jnp
from jax import lax
from jax.experimental import pallas as pl
from jax.experimental.pallas import tpu as pltpu
from jax.experimental.pallas import tpu_sc as plsc

NC = 2
NS = 16
K = 128
D = 128


def _mesh():
  return plsc.VectorSubcoreMesh(
      core_axis_name="c", subcore_axis_name="s", num_cores=NC,
      num_subcores=NS)


def _sc_degrees(idx_all, ones_k, zeros_d, n_pad):
  cpt = idx_all.shape[3]
  rpt = n_pad // NS

  @functools.partial(
      pl.kernel,
      out_type=jax.ShapeDtypeStruct((2, 2, n_pad, D), jnp.float32),
      mesh=_mesh(),
      scratch_types=[
          pltpu.VMEM((cpt, K), jnp.int32),
          pltpu.VMEM((K, D), jnp.float32),
          pltpu.VMEM_SHARED((n_pad, D), jnp.float32),
      ],
  )
  def k(idx_hbm, ones_hbm, zeros_hbm, out_hbm, eidx, onesb, acc):
    cid = lax.axis_index("c")
    sid = lax.axis_index("s")
    r0 = sid * rpt
    pltpu.sync_copy(ones_hbm, onesb)
    for dr in range(2):
      pltpu.sync_copy(idx_hbm.at[cid, dr, sid], eidx)
      pltpu.sync_copy(zeros_hbm, acc.at[pl.ds(r0, rpt)])
      plsc.subcore_barrier()

      def body(g, carry):
        pltpu.sync_copy(onesb, acc.at[eidx.at[g]], add=True)
        return carry

      lax.fori_loop(0, cpt, body, 0)
      plsc.subcore_barrier()
      pltpu.sync_copy(acc.at[pl.ds(r0, rpt)],
                      out_hbm.at[cid, dr, pl.ds(r0, rpt)])
      plsc.subcore_barrier()

  return k(idx_all, ones_k, zeros_d)


def _sc_segsum(h, idx_src, idx_dst, zeros_d, n_pad):
  cpt = idx_src.shape[2]
  rpt = n_pad // NS

  @functools.partial(
      pl.kernel,
      out_type=jax.ShapeDtypeStruct((2, n_pad, D), jnp.float32),
      mesh=_mesh(),
      scratch_types=[
          pltpu.VMEM((cpt, K), jnp.int32),
          pltpu.VMEM((cpt, K), jnp.int32),
          pltpu.VMEM((K, D), jnp.float32),
          pltpu.VMEM_SHARED((n_pad, D), jnp.float32),
          pltpu.SemaphoreType.DMA,
      ],
  )
  def k(h_hbm, src_hbm, dst_hbm, zeros_hbm, out_hbm, sidx, didx, rows0, acc,
        gsem0):
    cid = lax.axis_index("c")
    sid = lax.axis_index("s")
    r0 = sid * rpt
    pltpu.sync_copy(src_hbm.at[cid, sid], sidx)
    pltpu.sync_copy(dst_hbm.at[cid, sid], didx)
    pltpu.sync_copy(zeros_hbm, acc.at[pl.ds(r0, rpt)])
    plsc.subcore_barrier()

    def body(i, carry):
      pltpu.async_copy(h_hbm.at[sidx.at[i]], rows0, gsem0).wait()
      pltpu.sync_copy(rows0, acc.at[didx.at[i]], add=True)
      return carry

    lax.fori_loop(0, cpt, body, 0)
    plsc.subcore_barrier()
    pltpu.sync_copy(acc.at[pl.ds(r0, rpt)], out_hbm.at[cid, pl.ds(r0, rpt)])

  return k(h, idx_src, idx_dst, zeros_d)


def _scales(dg):
  so0 = lax.rsqrt(jnp.maximum(dg[0, 0, :, 0:1], 1.0))
  si0 = lax.rsqrt(jnp.maximum(dg[0, 1, :, 0:1], 1.0))
  so1 = lax.rsqrt(jnp.maximum(dg[1, 0, :, 0:1], 1.0))
  si1 = lax.rsqrt(jnp.maximum(dg[1, 1, :, 0:1], 1.0))
  return so0, si0, so1, si1


_BLK = 256


def _deg_spec():
  return pl.BlockSpec((2, 2, _BLK, D), lambda i: (0, 0, i, 0))


def _full(shape):
  return pl.BlockSpec(shape, lambda i: tuple(0 for _ in shape))


def _tc_init(x, w, b, deg, n_pad):
  def body(x_ref, w_ref, b_ref, deg_ref, h_ref):
    so0, _, so1, _ = _scales(deg_ref[...])
    x0 = jnp.dot(x_ref[...], w_ref[...],
                 preferred_element_type=jnp.float32) + b_ref[...]
    h_ref[...] = jnp.concatenate([x0 * so0, x0 * so1], axis=1)

  return pl.pallas_call(
      body,
      grid=(n_pad // _BLK,),
      in_specs=[
          pl.BlockSpec((_BLK, x.shape[1]), lambda i: (i, 0)),
          _full(w.shape),
          _full(b.shape),
          _deg_spec(),
      ],
      out_specs=pl.BlockSpec((_BLK, D), lambda i: (i, 0)),
      out_shape=jax.ShapeDtypeStruct((n_pad, D), jnp.float32),
  )(x, w, b, deg)


def _tc_mid(agg, w0, w1, b0, b1, deg, n_pad):
  def body(a_ref, w0_ref, w1_ref, b0_ref, b1_ref, deg_ref, h_ref):
    so0, si0, so1, si1 = _scales(deg_ref[...])
    a = a_ref[...]
    x = (jnp.dot(a[0][:, :64] * si0, w0_ref[...],
                 preferred_element_type=jnp.float32)
         + jnp.dot(a[1][:, 64:] * si1, w1_ref[...],
                   preferred_element_type=jnp.float32)
         + b0_ref[...] + b1_ref[...])
    h_ref[0] = x * so0
    h_ref[1] = x * so1

  return pl.pallas_call(
      body,
      grid=(n_pad // _BLK,),
      in_specs=[
          pl.BlockSpec((2, _BLK, D), lambda i: (0, i, 0)),
          _full(w0.shape), _full(w1.shape), _full(b0.shape), _full(b1.shape),
          _deg_spec(),
      ],
      out_specs=pl.BlockSpec((2, _BLK, D), lambda i: (0, i, 0)),
      out_shape=jax.ShapeDtypeStruct((2, n_pad, D), jnp.float32),
  )(agg, w0, w1, b0, b1, deg)


def _tc_l2(agg, w0, w1, b0, b1, w3_0, w3_1, deg, n_pad):
  def body(a_ref, w0_ref, w1_ref, b0_ref, b1_ref, w30_ref, w31_ref, deg_ref,
           z_ref):
    so0, si0, so1, si1 = _scales(deg_ref[...])
    a = a_ref[...]
    x = (jnp.dot(a[0] * si0, w0_ref[...], preferred_element_type=jnp.float32)
         + jnp.dot(a[1] * si1, w1_ref[...], preferred_element_type=jnp.float32)
         + b0_ref[...] + b1_ref[...])
    z0 = jnp.dot(x * so0, w30_ref[...], preferred_element_type=jnp.float32)
    z1 = jnp.dot(x * so1, w31_ref[...], preferred_element_type=jnp.float32)
    z_ref[...] = jnp.concatenate([z0, z1], axis=1)

  return pl.pallas_call(
      body,
      grid=(n_pad // _BLK,),
      in_specs=[
          pl.BlockSpec((2, _BLK, D), lambda i: (0, i, 0)),
          _full(w0.shape), _full(w1.shape), _full(b0.shape), _full(b1.shape),
          _full(w3_0.shape), _full(w3_1.shape),
          _deg_spec(),
      ],
      out_specs=pl.BlockSpec((_BLK, D), lambda i: (i, 0)),
      out_shape=jax.ShapeDtypeStruct((n_pad, D), jnp.float32),
  )(agg, w0, w1, b0, b1, w3_0, w3_1, deg)


def _tc_final(agg, b0, b1, deg, n):
  def body(a_ref, b0_ref, b1_ref, deg_ref, o_ref):
    _, si0, _, si1 = _scales(deg_ref[...])
    a = a_ref[...]
    o_ref[...] = (a[0][:, :64] * si0 + a[1][:, 64:] * si1
                  + b0_ref[...] + b1_ref[...])

  return pl.pallas_call(
      body,
      grid=(agg.shape[1] // _BLK,),
      in_specs=[
          pl.BlockSpec((2, _BLK, D), lambda i: (0, i, 0)),
          _full(b0.shape), _full(b1.shape),
          _deg_spec(),
      ],
      out_specs=pl.BlockSpec((_BLK, 64), lambda i: (i, 0)),
      out_shape=jax.ShapeDtypeStruct((n, 64), jnp.float32),
  )(agg, b0, b1, deg)


def kernel(input_data, edge_index_rel0, edge_index_rel1, W_init, b_init,
           W1_0, b1_0, W1_1, b1_1, W2_0, b2_0, W2_1, b2_1,
           W3_0, b3_0, W3_1, b3_1):
  n = input_data.shape[0]
  e = edge_index_rel0.shape[1]
  n_pad = ((n + _BLK - 1) // _BLK) * _BLK
  ept = e // NS
  ept_pad = ((ept + K - 1) // K) * K
  cpt = ept_pad // K
  n_sent = ept_pad - ept
  assert e % NS == 0 and n + n_sent <= n_pad

  idx = jnp.stack([edge_index_rel0, edge_index_rel1]).astype(
      jnp.int32).reshape(2, 2, NS, ept)
  sent = jnp.broadcast_to(
      (n + jnp.arange(n_sent, dtype=jnp.int32))[None, None, None, :],
      (2, 2, NS, n_sent))
  idx_all = jnp.concatenate([idx, sent], axis=3).reshape(2, 2, NS, cpt, K)
  idx_src = idx_all[:, 0]
  idx_dst = idx_all[:, 1]
  idx_src2 = idx_src + (jnp.arange(2, dtype=jnp.int32) * n_pad)[
      :, None, None, None]

  zeros_d = jnp.zeros((n_pad // NS, D), jnp.float32)
  ones_k = jnp.ones((K, D), jnp.float32)

  deg = _sc_degrees(idx_all, ones_k, zeros_d, n_pad)
  h1 = _tc_init(input_data, W_init, b_init.reshape(1, -1), deg, n_pad)
  agg1 = _sc_segsum(h1, idx_src, idx_dst, zeros_d, n_pad)
  h2 = _tc_mid(agg1, W1_0, W1_1, b1_0.reshape(1, -1), b1_1.reshape(1, -1),
               deg, n_pad)
  agg2 = _sc_segsum(h2.reshape(2 * n_pad, D), idx_src2, idx_dst, zeros_d,
                    n_pad)
  z3 = _tc_l2(agg2, W2_0, W2_1, b2_0.reshape(1, -1), b2_1.reshape(1, -1),
              W3_0, W3_1, deg, n_pad)
  agg3 = _sc_segsum(z3, idx_src, idx_dst, zeros_d, n_pad)
  return _tc_final(agg3, b3_0.reshape(1, -1), b3_1.reshape(1, -1), deg, n)
